# baseline (speedup 1.0000x reference)
import functools

import jax
import jax.numpy as jnp
from jax import lax
from jax.experimental import pallas as pl
from jax.experimental.pallas import tpu as pltpu
from jax.experimental.pallas import tpu_sc as plsc

B, L, V = 1024, 50, 1000
N = B * L
NC, NS, LANES = 2, 16, 16
NW = NC * NS
PER_W = N // NW
CHUNK = 64
NCHUNK = PER_W // CHUNK

@functools.cache
def _make_sc_gather():
    mesh = plsc.VectorSubcoreMesh(core_axis_name="c", subcore_axis_name="s")
    return functools.partial(
        pl.kernel,
        mesh=mesh,
        out_type=[
            jax.ShapeDtypeStruct((N,), jnp.float32),
            jax.ShapeDtypeStruct((N,), jnp.float32),
        ],
        scratch_types=[
            pltpu.VMEM((PER_W,), jnp.int32),
            pltpu.VMEM((PER_W,), jnp.int32),
            pltpu.VMEM((PER_W,), jnp.int32),
            pltpu.VMEM((PER_W,), jnp.int32),
            pltpu.VMEM((PER_W,), jnp.float32),
            pltpu.VMEM((PER_W,), jnp.float32),
            pltpu.SemaphoreType.DMA,
        ],
    )(_sc_gather_body)


def _sc_gather_body(scores_hbm, pos_hbm, neg_hbm, pout_hbm, nout_hbm,
                    pidx_v, nidx_v, pflat_v, nflat_v, pg_v, ng_v, sem):
    wid = lax.axis_index("s") * NC + lax.axis_index("c")
    base = wid * PER_W

    pltpu.sync_copy(pos_hbm.at[pl.ds(base, PER_W)], pidx_v)
    pltpu.sync_copy(neg_hbm.at[pl.ds(base, PER_W)], nidx_v)

    def _flatten(j, carry):
        off = j * LANES
        gi = (base + off + lax.iota(jnp.int32, LANES)) * V
        pflat_v[pl.ds(off, LANES)] = gi + pidx_v[pl.ds(off, LANES)]
        nflat_v[pl.ds(off, LANES)] = gi + nidx_v[pl.ds(off, LANES)]
        return carry

    lax.fori_loop(0, PER_W // LANES, _flatten, 0)

    def _fire(c, carry):
        o = c * CHUNK
        pltpu.make_async_copy(
            scores_hbm.at[pflat_v.at[pl.ds(o, CHUNK)]],
            pg_v.at[pl.ds(o, CHUNK)], sem).start()
        pltpu.make_async_copy(
            scores_hbm.at[nflat_v.at[pl.ds(o, CHUNK)]],
            ng_v.at[pl.ds(o, CHUNK)], sem).start()
        return carry

    lax.fori_loop(0, NCHUNK, _fire, 0)

    def _drain(c, carry):
        o = c * CHUNK
        pltpu.make_async_copy(
            scores_hbm.at[pflat_v.at[pl.ds(o, CHUNK)]],
            pg_v.at[pl.ds(o, CHUNK)], sem).wait()
        pltpu.make_async_copy(
            scores_hbm.at[nflat_v.at[pl.ds(o, CHUNK)]],
            ng_v.at[pl.ds(o, CHUNK)], sem).wait()
        return carry

    lax.fori_loop(0, NCHUNK, _drain, 0)

    pltpu.sync_copy(pg_v, pout_hbm.at[pl.ds(base, PER_W)])
    pltpu.sync_copy(ng_v, nout_hbm.at[pl.ds(base, PER_W)])


def _loss_body(pg_ref, ng_ref, w_ref, pi_ref, ni_ref, out_ref):
    pg = pg_ref[...]
    ng = ng_ref[...]
    w = w_ref[...]
    valid = jnp.logical_and(pi_ref[...] > 0, ni_ref[...] > 0)
    x = (pg - ng) * w
    ls = jnp.minimum(x, 0.0) - jnp.log1p(jnp.exp(-jnp.abs(x)))
    v = valid.astype(jnp.float32)
    s = jnp.sum(ls * v)
    c = jnp.sum(v)
    loss = -jnp.where(c == 0.0, 0.0, s / jnp.maximum(c, 1.0))
    out_ref[...] = jnp.zeros((1, 1), jnp.float32) + loss


_tc_loss = pl.pallas_call(
    _loss_body,
    out_shape=jax.ShapeDtypeStruct((1, 1), jnp.float32),
)


def kernel(scores, positive_mask, negative_mask, weights):
    pos = positive_mask.reshape(-1).astype(jnp.int32)
    neg = negative_mask.reshape(-1).astype(jnp.int32)
    flat_scores = scores.reshape(-1)
    pg, ng = _make_sc_gather()(flat_scores, pos, neg)
    R, C = 400, 128
    loss = _tc_loss(
        pg.reshape(R, C), ng.reshape(R, C), weights.reshape(R, C),
        pos.reshape(R, C), neg.reshape(R, C))
    return loss.reshape(())


# speedup vs baseline: 1.5223x; 1.5223x over previous
import functools

import jax
import jax.numpy as jnp
from jax import lax
from jax.experimental import pallas as pl
from jax.experimental.pallas import tpu as pltpu
from jax.experimental.pallas import tpu_sc as plsc

B, L, V = 1024, 50, 1000
N = B * L
NC, NS, LANES = 2, 16, 16
NW = NC * NS
NB_W = B // NW
PER_W = NB_W * L
GROUPS = (0, 16, 32, 34)


@functools.cache
def _make_sc_gather():
    mesh = plsc.VectorSubcoreMesh(core_axis_name="c", subcore_axis_name="s")
    return functools.partial(
        pl.kernel,
        mesh=mesh,
        out_type=[
            jax.ShapeDtypeStruct((N,), jnp.float32),
            jax.ShapeDtypeStruct((N,), jnp.float32),
        ],
        scratch_types=[
            pltpu.VMEM((PER_W,), jnp.int32),
            pltpu.VMEM((PER_W,), jnp.int32),
            pltpu.VMEM((L, V), jnp.float32),
            pltpu.VMEM((L, V), jnp.float32),
            pltpu.VMEM((PER_W,), jnp.float32),
            pltpu.VMEM((PER_W,), jnp.float32),
            pltpu.SemaphoreType.DMA,
            pltpu.SemaphoreType.DMA,
        ],
        compiler_params=pltpu.CompilerParams(
            use_tc_tiling_on_sc=True,
            disable_bounds_checks=True,
        ),
    )(_sc_gather_body)


def _sc_gather_body(scores_hbm, pos_hbm, neg_hbm, pout_hbm, nout_hbm,
                    pidx_v, nidx_v, buf0, buf1, pg_v, ng_v, sem0, sem1):
    wid = lax.axis_index("s") * NC + lax.axis_index("c")
    base = wid * PER_W
    b0 = wid * NB_W

    pltpu.sync_copy(pos_hbm.at[pl.ds(base, PER_W)], pidx_v)
    pltpu.sync_copy(neg_hbm.at[pl.ds(base, PER_W)], nidx_v)

    def _extract(buf, bi):
        off = bi * L
        lane = lax.iota(jnp.int32, LANES)
        for g in GROUPS:
            pi16 = pidx_v[pl.ds(off + g, LANES)]
            ni16 = nidx_v[pl.ds(off + g, LANES)]
            pc0 = pi16 & ~(LANES - 1)
            nc0 = ni16 & ~(LANES - 1)
            pcm = pi16 & (LANES - 1)
            ncm = ni16 & (LANES - 1)
            pacc = jnp.zeros((LANES,), jnp.float32)
            nacc = jnp.zeros((LANES,), jnp.float32)
            for k in range(LANES):
                row = g + k
                pv = buf[row, pl.ds(pl.multiple_of(pc0[k], LANES), LANES)]
                nv = buf[row, pl.ds(pl.multiple_of(nc0[k], LANES), LANES)]
                ps = jnp.take(pv, pcm)[k]
                ns = jnp.take(nv, ncm)[k]
                pacc = jnp.where(lane == k, ps, pacc)
                nacc = jnp.where(lane == k, ns, nacc)
            pg_v[pl.ds(off + g, LANES)] = pacc
            ng_v[pl.ds(off + g, LANES)] = nacc

    pltpu.make_async_copy(scores_hbm.at[b0], buf0, sem0).start()

    def _step(bi, carry):
        nxt = bi + 1

        @pl.when(nxt < NB_W)
        def _prefetch():
            @pl.when(nxt % 2 == 0)
            def _():
                pltpu.make_async_copy(
                    scores_hbm.at[b0 + nxt], buf0, sem0).start()

            @pl.when(nxt % 2 == 1)
            def _():
                pltpu.make_async_copy(
                    scores_hbm.at[b0 + nxt], buf1, sem1).start()

        @pl.when(bi % 2 == 0)
        def _use0():
            pltpu.make_async_copy(scores_hbm.at[b0], buf0, sem0).wait()
            _extract(buf0, bi)

        @pl.when(bi % 2 == 1)
        def _use1():
            pltpu.make_async_copy(scores_hbm.at[b0], buf1, sem1).wait()
            _extract(buf1, bi)

        return carry

    lax.fori_loop(0, NB_W, _step, 0)

    pltpu.sync_copy(pg_v, pout_hbm.at[pl.ds(base, PER_W)])
    pltpu.sync_copy(ng_v, nout_hbm.at[pl.ds(base, PER_W)])


def _loss_body(pg_ref, ng_ref, w_ref, pi_ref, ni_ref, out_ref):
    pg = pg_ref[...]
    ng = ng_ref[...]
    w = w_ref[...]
    valid = jnp.logical_and(pi_ref[...] > 0, ni_ref[...] > 0)
    x = (pg - ng) * w
    ls = jnp.minimum(x, 0.0) - jnp.log1p(jnp.exp(-jnp.abs(x)))
    v = valid.astype(jnp.float32)
    s = jnp.sum(ls * v)
    c = jnp.sum(v)
    loss = -jnp.where(c == 0.0, 0.0, s / jnp.maximum(c, 1.0))
    out_ref[...] = jnp.zeros((1, 1), jnp.float32) + loss


_tc_loss = pl.pallas_call(
    _loss_body,
    out_shape=jax.ShapeDtypeStruct((1, 1), jnp.float32),
)


def kernel(scores, positive_mask, negative_mask, weights):
    pos = positive_mask.reshape(-1).astype(jnp.int32)
    neg = negative_mask.reshape(-1).astype(jnp.int32)
    pg, ng = _make_sc_gather()(scores, pos, neg)
    R, C = 400, 128
    loss = _tc_loss(
        pg.reshape(R, C), ng.reshape(R, C), weights.reshape(R, C),
        pos.reshape(R, C), neg.reshape(R, C))
    return loss.reshape(())
